# Initial kernel scaffold; baseline (speedup 1.0000x reference)
import jax
import jax.numpy as jnp
from jax.experimental import pallas as pl


def kernel(x, edge_index, node_index, W1, b1, W2, b2, W3, b3, Wp, bp, Wa, ba, Wm, bm, Wg, bg, Wt, bt):
    raise NotImplementedError("write your pallas kernel here")


# speedup vs baseline: 25.8624x; 25.8624x over previous
import functools

import jax
import jax.numpy as jnp
from jax import lax
from jax.experimental import pallas as pl
from jax.experimental.pallas import tpu as pltpu
from jax.experimental.pallas import tpu_sc as plsc

_NC = 2
_NS = 16
_L = 16

_N = 50000
_E = 800000
_H = 64
_HH = 32
_NPAD = 50048
_JUNK = 50000
_EPAD = 802816
_ER = 6272
_RPT = 392
_CH = 8
_NCHUNK = 49
_DEG_CH = 2048
_DEG_PT = 25088
_TB = 400


def _sc_mesh():
    return plsc.VectorSubcoreMesh(
        core_axis_name="c", subcore_axis_name="s",
        num_cores=_NC, num_subcores=_NS)


def _sc_deg_cnt(srcp, dstp, ni16):
    @functools.partial(
        pl.kernel,
        out_type=(
            jax.ShapeDtypeStruct((_NC * _NS * _NPAD,), jnp.float32),
            jax.ShapeDtypeStruct((_NC * _NS * _NPAD,), jnp.float32),
        ),
        mesh=_sc_mesh(),
        compiler_params=pltpu.CompilerParams(needs_layout_passes=False),
        scratch_types=[
            pltpu.VMEM((_NPAD,), jnp.float32),
            pltpu.VMEM((_NPAD,), jnp.float32),
            pltpu.VMEM((_DEG_CH,), jnp.int32),
            pltpu.VMEM((_DEG_CH,), jnp.int32),
            pltpu.VMEM((_L,), jnp.int32),
        ],
    )
    def k(src_h, dst_h, ni_h, deg_o, cnt_o, degp, cntp, sbuf, dbuf, nibuf):
        c = lax.axis_index("c")
        s = lax.axis_index("s")
        t = c * _NS + s
        pltpu.sync_copy(ni_h, nibuf)
        niv = nibuf[...]
        zeros = jnp.zeros((_L,), jnp.float32)
        ones = jnp.ones((_L,), jnp.float32)

        def zero_body(i, _):
            degp[pl.ds(i * _L, _L)] = zeros
            cntp[pl.ds(i * _L, _L)] = zeros
            return 0
        lax.fori_loop(0, _NPAD // _L, zero_body, 0)

        ebase = t * _DEG_PT

        def accum(nvec):
            def body(j, _):
                sv = sbuf[pl.ds(j * _L, _L)]
                dv = dbuf[pl.ds(j * _L, _L)]
                plsc.addupdate_scatter(degp, [dv], ones)
                cv = jnp.where(dv == niv, 1.0, 0.0).astype(jnp.float32)
                plsc.addupdate_scatter(cntp, [sv], cv)
                return 0
            lax.fori_loop(0, nvec, body, 0)

        nfull = _DEG_PT // _DEG_CH
        tail = _DEG_PT - nfull * _DEG_CH

        def chunk(kk, _):
            off = ebase + kk * _DEG_CH
            pltpu.sync_copy(src_h.at[pl.ds(off, _DEG_CH)], sbuf)
            pltpu.sync_copy(dst_h.at[pl.ds(off, _DEG_CH)], dbuf)
            accum(_DEG_CH // _L)
            return 0
        lax.fori_loop(0, nfull, chunk, 0)

        off = ebase + nfull * _DEG_CH
        pltpu.sync_copy(src_h.at[pl.ds(off, tail)], sbuf.at[pl.ds(0, tail)])
        pltpu.sync_copy(dst_h.at[pl.ds(off, tail)], dbuf.at[pl.ds(0, tail)])
        accum(tail // _L)

        pltpu.sync_copy(degp, deg_o.at[pl.ds(t * _NPAD, _NPAD)])
        pltpu.sync_copy(cntp, cnt_o.at[pl.ds(t * _NPAD, _NPAD)])

    return k(srcp, dstp, ni16)


def _sc_aggregate(src2, dst2, xs_lo, xs_hi):
    @functools.partial(
        pl.kernel,
        out_type=jax.ShapeDtypeStruct((_NC, _NPAD, _HH), jnp.float32),
        mesh=_sc_mesh(),
        compiler_params=pltpu.CompilerParams(use_tc_tiling_on_sc=False),
        scratch_types=[
            pltpu.VMEM((_CH, 128), jnp.int32),
            pltpu.VMEM((_CH, 128), jnp.int32),
            pltpu.VMEM((4, 128, _HH), jnp.float32),
            pltpu.VMEM_SHARED((_NPAD, _HH), jnp.float32),
            pltpu.SemaphoreType.DMA,
            pltpu.SemaphoreType.DMA,
        ],
    )
    def k(src_h, dst_h, xlo_h, xhi_h, out_h,
          sidx, didx, rows, acc, gsem, ssem):
        c = lax.axis_index("c")
        s = lax.axis_index("s")

        zeros = jnp.zeros((_L,), jnp.float32)
        for g in range(4):
            def zb(i, _):
                rows[g, i, pl.ds(0, _L)] = zeros
                rows[g, i, pl.ds(_L, _L)] = zeros
                return 0
            lax.fori_loop(0, 128, zb, 0)
        zrow = s * (_NPAD // _NS)

        def za(i, _):
            pltpu.sync_copy(rows.at[0], acc.at[pl.ds(zrow + i * 128, 128)])
            return 0
        lax.fori_loop(0, 24, za, 0)
        pltpu.sync_copy(rows.at[0].at[pl.ds(0, 56)],
                        acc.at[pl.ds(zrow + 3072, 56)])
        plsc.subcore_barrier()

        def edge_pass(xs_ref):
            rbase = s * _RPT

            def half(lo):
                def fg(g, _):
                    pltpu.async_copy(
                        xs_ref.at[sidx.at[lo + g]], rows.at[g], gsem)
                    return 0
                lax.fori_loop(0, 4, fg, 0)

                def dg(g, _):
                    pltpu.make_async_copy(
                        xs_ref.at[sidx.at[lo + g]], rows.at[g], gsem).wait()
                    return 0
                lax.fori_loop(0, 4, dg, 0)

                def fs(g, _):
                    pltpu.async_copy(
                        rows.at[g], acc.at[didx.at[lo + g]], ssem, add=True)
                    return 0
                lax.fori_loop(0, 4, fs, 0)

                def dr(g, _):
                    pltpu.make_async_copy(
                        rows.at[g], acc.at[didx.at[lo + g]], ssem).wait()
                    return 0
                lax.fori_loop(0, 4, dr, 0)

            def chunk(kk, _):
                off = rbase + kk * _CH
                pltpu.sync_copy(src_h.at[pl.ds(off, _CH)], sidx)
                pltpu.sync_copy(dst_h.at[pl.ds(off, _CH)], didx)
                half(0)
                half(4)
                return 0

            lax.fori_loop(0, _NCHUNK, chunk, 0)

        @pl.when(c == 0)
        def _():
            edge_pass(xlo_h)

        @pl.when(c == 1)
        def _():
            edge_pass(xhi_h)

        plsc.subcore_barrier()

        wrow = s * (_NPAD // _NS)

        @pl.when(c == 0)
        def _():
            pltpu.sync_copy(acc.at[pl.ds(wrow, _NPAD // _NS)],
                            out_h.at[0].at[pl.ds(wrow, _NPAD // _NS)])

        @pl.when(c == 1)
        def _():
            pltpu.sync_copy(acc.at[pl.ds(wrow, _NPAD // _NS)],
                            out_h.at[1].at[pl.ds(wrow, _NPAD // _NS)])

    return k(src2, dst2, xs_lo, xs_hi)


def _tc1(deg_pt, cnt_pt, xp, w1p):
    def body(deg_r, cnt_r, x_r, w_r, dinv_r, ap_r, lo_r, hi_r):
        deg = jnp.sum(deg_r[...], axis=1, keepdims=True) + 1.0
        dinv2 = lax.rsqrt(jnp.maximum(deg, 1.0))
        cnt = jnp.sum(cnt_r[...], axis=1, keepdims=True)
        dinv_r[...] = dinv2
        ap_r[...] = cnt * dinv2
        xw = jnp.dot(x_r[...], w_r[...], preferred_element_type=jnp.float32)
        xs = xw * dinv2
        lo_r[...] = xs[:, :_HH]
        hi_r[...] = xs[:, _HH:]

    grid = _N // _TB
    return pl.pallas_call(
        body,
        grid=(grid,),
        in_specs=[
            pl.BlockSpec((_TB, _NC * _NS), lambda i: (i, 0)),
            pl.BlockSpec((_TB, _NC * _NS), lambda i: (i, 0)),
            pl.BlockSpec((_TB, 128), lambda i: (i, 0)),
            pl.BlockSpec((128, _H), lambda i: (0, 0)),
        ],
        out_specs=[
            pl.BlockSpec((_TB, 1), lambda i: (i, 0)),
            pl.BlockSpec((_TB, 1), lambda i: (i, 0)),
            pl.BlockSpec((_TB, _HH), lambda i: (i, 0)),
            pl.BlockSpec((_TB, _HH), lambda i: (i, 0)),
        ],
        out_shape=(
            jax.ShapeDtypeStruct((_N, 1), jnp.float32),
            jax.ShapeDtypeStruct((_N, 1), jnp.float32),
            jax.ShapeDtypeStruct((_N, _HH), jnp.float32),
            jax.ShapeDtypeStruct((_N, _HH), jnp.float32),
        ),
    )(deg_pt, cnt_pt, xp, w1p)


def _tc2(agg1, xs_lo, xs_hi, dinv, w2, b1):
    def body(a_r, lo_r, hi_r, dinv_r, w_r, b_r, olo_r, ohi_r):
        t = jnp.concatenate([a_r[0] + lo_r[...], a_r[1] + hi_r[...]], axis=1)
        dinv2 = dinv_r[...]
        h1 = jnp.maximum(dinv2 * t + b_r[...], 0.0)
        y2 = jnp.dot(h1, w_r[...], preferred_element_type=jnp.float32)
        y2s = dinv2 * y2
        olo_r[...] = y2s[:, :_HH]
        ohi_r[...] = y2s[:, _HH:]

    grid = _N // _TB
    return pl.pallas_call(
        body,
        grid=(grid,),
        in_specs=[
            pl.BlockSpec((_NC, _TB, _HH), lambda i: (0, i, 0)),
            pl.BlockSpec((_TB, _HH), lambda i: (i, 0)),
            pl.BlockSpec((_TB, _HH), lambda i: (i, 0)),
            pl.BlockSpec((_TB, 1), lambda i: (i, 0)),
            pl.BlockSpec((_H, _H), lambda i: (0, 0)),
            pl.BlockSpec((1, _H), lambda i: (0, 0)),
        ],
        out_specs=[
            pl.BlockSpec((_TB, _HH), lambda i: (i, 0)),
            pl.BlockSpec((_TB, _HH), lambda i: (i, 0)),
        ],
        out_shape=(
            jax.ShapeDtypeStruct((_N, _HH), jnp.float32),
            jax.ShapeDtypeStruct((_N, _HH), jnp.float32),
        ),
    )(agg1, xs_lo, xs_hi, dinv, w2, b1)


def _tc3(agg2, y2s_lo, y2s_hi, dinv, ap, ni_arr, b2, w3, b3, wp, bp, wh, bh):
    grid = _N // _TB

    def body(ni_r, a_r, lo_r, hi_r, dinv_r, ap_r, b2_r, w3_r, b3_r,
             wp_r, bp_r, wh_r, bh_r, out_r, s1_acc, hni_acc, dni_acc):
        i = pl.program_id(0)

        @pl.when(i == 0)
        def _():
            s1_acc[...] = jnp.zeros_like(s1_acc)
            hni_acc[...] = jnp.zeros_like(hni_acc)
            dni_acc[...] = jnp.zeros_like(dni_acc)

        t = jnp.concatenate([a_r[0] + lo_r[...], a_r[1] + hi_r[...]], axis=1)
        dinv2 = dinv_r[...]
        h2 = jnp.maximum(dinv2 * t + b2_r[...], 0.0)
        s1_acc[...] += jnp.sum(ap_r[...] * h2, axis=0, keepdims=True)

        ni = ni_r[0]
        rows = i * _TB + lax.broadcasted_iota(jnp.int32, (_TB, 1), 0)
        msel = rows == ni
        dni_acc[...] += jnp.sum(jnp.where(msel, dinv2, 0.0),
                                axis=0, keepdims=True)
        hni_acc[...] += jnp.sum(jnp.where(msel, h2, 0.0),
                                axis=0, keepdims=True)

        @pl.when(i == grid - 1)
        def _():
            dni = dni_acc[0, 0]
            u = dni * (s1_acc[...] + dni * hni_acc[...])
            u8 = jnp.broadcast_to(u, (8, _H))
            h3 = jnp.maximum(
                jnp.dot(u8, w3_r[...], preferred_element_type=jnp.float32)
                + b3_r[...], 0.0)
            nd = jnp.maximum(
                jnp.dot(h3, wp_r[...], preferred_element_type=jnp.float32)
                + bp_r[...], 0.0)
            out_r[...] = (jnp.dot(nd, wh_r[...],
                                  preferred_element_type=jnp.float32)
                          + bh_r[...])

    return pl.pallas_call(
        body,
        grid=(grid,),
        in_specs=[
            pl.BlockSpec(memory_space=pltpu.SMEM),
            pl.BlockSpec((_NC, _TB, _HH), lambda i: (0, i, 0)),
            pl.BlockSpec((_TB, _HH), lambda i: (i, 0)),
            pl.BlockSpec((_TB, _HH), lambda i: (i, 0)),
            pl.BlockSpec((_TB, 1), lambda i: (i, 0)),
            pl.BlockSpec((_TB, 1), lambda i: (i, 0)),
            pl.BlockSpec((1, _H), lambda i: (0, 0)),
            pl.BlockSpec((_H, _H), lambda i: (0, 0)),
            pl.BlockSpec((1, _H), lambda i: (0, 0)),
            pl.BlockSpec((_H, _H), lambda i: (0, 0)),
            pl.BlockSpec((1, _H), lambda i: (0, 0)),
            pl.BlockSpec((_H, 128), lambda i: (0, 0)),
            pl.BlockSpec((1, 128), lambda i: (0, 0)),
        ],
        out_specs=pl.BlockSpec((8, 128), lambda i: (0, 0)),
        out_shape=jax.ShapeDtypeStruct((8, 128), jnp.float32),
        scratch_shapes=[
            pltpu.VMEM((1, _H), jnp.float32),
            pltpu.VMEM((1, _H), jnp.float32),
            pltpu.VMEM((1, 1), jnp.float32),
        ],
    )(ni_arr, agg2, y2s_lo, y2s_hi, dinv, ap, b2, w3, b3, wp, bp, wh, bh)


def kernel(x, edge_index, node_index, W1, b1, W2, b2, W3, b3,
           Wp, bp, Wa, ba, Wm, bm, Wg, bg, Wt, bt):
    ni = jnp.asarray(node_index, jnp.int32)

    ei = edge_index.astype(jnp.int32)
    src = jnp.concatenate(
        [ei[0], jnp.zeros((_EPAD - _E,), jnp.int32)])
    dst = jnp.concatenate(
        [ei[1], jnp.full((_EPAD - _E,), _JUNK, jnp.int32)])
    src2 = src.reshape(_ER, 128)
    dst2 = dst.reshape(_ER, 128)
    ni16 = jnp.full((_L,), ni, jnp.int32)

    deg_p, cnt_p = _sc_deg_cnt(src, dst, ni16)

    xp = jnp.pad(x, ((0, 0), (0, 128 - x.shape[1])))
    w1p = jnp.pad(W1, ((0, 128 - W1.shape[0]), (0, 0)))
    dinv, ap, xs_lo, xs_hi = _tc1(deg_p.reshape(_NC * _NS, _NPAD).T,
                                  cnt_p.reshape(_NC * _NS, _NPAD).T, xp, w1p)

    agg1 = _sc_aggregate(src2, dst2, xs_lo, xs_hi)
    y2s_lo, y2s_hi = _tc2(agg1, xs_lo, xs_hi, dinv, W2, b1.reshape(1, _H))

    agg2 = _sc_aggregate(src2, dst2, y2s_lo, y2s_hi)

    wh = jnp.pad(jnp.concatenate([Wa, Wm, Wg, Wt], axis=1),
                 ((0, 0), (0, 128 - 31)))
    bh = jnp.pad(jnp.concatenate([ba, bm, bg, bt]), (0, 128 - 31))
    out = _tc3(agg2, y2s_lo, y2s_hi, dinv, ap, ni.reshape(1),
               b2.reshape(1, _H), W3, b3.reshape(1, _H),
               Wp, bp.reshape(1, _H), wh, bh.reshape(1, 128))

    o = out[0]
    return (o[:6], o[6:8], o[8:11], o[11:31])
